# baseline (speedup 1.0000x reference)
import functools
import math

import jax
import jax.numpy as jnp
from jax import lax
from jax.experimental import pallas as pl
from jax.experimental.pallas import tpu as pltpu
from jax.experimental.pallas import tpu_sc as plsc

N = 10000
E = 320000
H = 128
CH = 128
NTILES = 32
EPT = E // NTILES
NA = 10112
NCH = 81
EPP = NCH * CH
RTZ = NA // 16
RW = 632
DEPTH = 3

_f32 = jnp.float32


def _sc_mesh():
    return plsc.VectorSubcoreMesh(core_axis_name="c", subcore_axis_name="s",
                                  num_cores=2, num_subcores=16)


def _sc_deg_body(dst_hbm, out_hbm, didx, cnt, sem):
    cid = lax.axis_index("c")
    sid = lax.axis_index("s")
    wid = cid * 16 + sid

    def zrow(i, c):
        cnt[pl.ds(i * 16, 16)] = jnp.zeros((16,), _f32)
        return c

    lax.fori_loop(0, N // 16, zrow, 0)
    pltpu.sync_copy(dst_hbm.at[pl.ds(wid * EPT, EPT)], didx)
    ones = jnp.ones((16,), _f32)

    def grp(g, c):
        idx = didx[pl.ds(g * 16, 16)]
        plsc.addupdate_scatter(cnt, [idx], ones)
        return c

    lax.fori_loop(0, EPT // 16, grp, 0)
    pltpu.sync_copy(cnt, out_hbm.at[pl.ds(wid * N, N)])


@functools.lru_cache(maxsize=None)
def _sc_deg_kernel():
    return pl.kernel(
        _sc_deg_body,
        out_type=jax.ShapeDtypeStruct((NTILES * N,), _f32),
        mesh=_sc_mesh(),
        compiler_params=pltpu.CompilerParams(needs_layout_passes=False),
        scratch_types=[
            pltpu.VMEM((EPT,), jnp.int32),
            pltpu.VMEM((N,), _f32),
            pltpu.SemaphoreType.DMA,
        ],
    )


def _sc_agg_body(g_hbm, src_hbm, dst_hbm, zero_hbm, out0_hbm, out1_hbm,
                 sidx0, sidx1, sidx2, didx0, didx1, didx2,
                 rows0, rows1, rows2, acc,
                 gs0, gs1, gs2, ss0, ss1, ss2):
    cid = lax.axis_index("c")
    sid = lax.axis_index("s")
    wid = cid * 16 + sid
    r0 = sid * RTZ
    base = wid * EPP
    pltpu.sync_copy(zero_hbm.at[pl.ds(r0, RTZ)], acc.at[pl.ds(r0, RTZ)])
    plsc.subcore_barrier()

    S = ((sidx0, didx0, rows0, gs0, ss0), (sidx1, didx1, rows1, gs1, ss1),
         (sidx2, didx2, rows2, gs2, ss2))

    def load_and_gather(off, slot):
        si, di, rw, gs, _ = slot
        pltpu.sync_copy(src_hbm.at[pl.ds(off, CH)], si)
        pltpu.sync_copy(dst_hbm.at[pl.ds(off, CH)], di)
        pltpu.async_copy(g_hbm.at[si], rw, gs)

    def wait_gather(slot):
        si, _, rw, gs, _ = slot
        pltpu.make_async_copy(g_hbm.at[si], rw, gs).wait()

    def start_scatter(slot):
        _, di, rw, _, ss = slot
        pltpu.async_copy(rw, acc.at[di], ss, add=True)

    def wait_scatter(slot):
        _, di, rw, _, ss = slot
        pltpu.make_async_copy(rw, acc.at[di], ss).wait()

    for j in range(DEPTH):
        load_and_gather(base + j * CH, S[j])
        if j > 0:
            wait_gather(S[j - 1])
            start_scatter(S[j - 1])

    def step(k, carry):
        for b in range(DEPTH):
            j = DEPTH * k + b
            slot, prev = S[b], S[b - 1]
            wait_scatter(slot)
            off = pl.multiple_of(base + j * CH, CH)
            load_and_gather(off, slot)
            wait_gather(prev)
            start_scatter(prev)
        return carry

    lax.fori_loop(1, NCH // DEPTH, step, 0)

    last = S[(NCH - 1) % DEPTH]
    wait_gather(last)
    start_scatter(last)
    for b in range(DEPTH):
        wait_scatter(S[b])
    plsc.subcore_barrier()
    rw0 = sid * RW

    @pl.when((cid == 0) & (sid < 15))
    def _w0():
        pltpu.sync_copy(acc.at[pl.ds(rw0, RW)], out0_hbm.at[pl.ds(rw0, RW)])

    @pl.when((cid == 0) & (sid == 15))
    def _w0t():
        pltpu.sync_copy(acc.at[pl.ds(15 * RW, N - 15 * RW)],
                        out0_hbm.at[pl.ds(15 * RW, N - 15 * RW)])

    @pl.when((cid == 1) & (sid < 15))
    def _w1():
        pltpu.sync_copy(acc.at[pl.ds(rw0, RW)], out1_hbm.at[pl.ds(rw0, RW)])

    @pl.when((cid == 1) & (sid == 15))
    def _w1t():
        pltpu.sync_copy(acc.at[pl.ds(15 * RW, N - 15 * RW)],
                        out1_hbm.at[pl.ds(15 * RW, N - 15 * RW)])


@functools.lru_cache(maxsize=None)
def _sc_agg_kernel():
    return pl.kernel(
        _sc_agg_body,
        out_type=(jax.ShapeDtypeStruct((N, H), _f32),
                  jax.ShapeDtypeStruct((N, H), _f32)),
        mesh=_sc_mesh(),
        scratch_types=(
            [pltpu.VMEM((CH,), jnp.int32)] * (2 * DEPTH)
            + [pltpu.VMEM((CH, H), _f32)] * DEPTH
            + [pltpu.VMEM_SHARED((NA, H), _f32)]
            + [pltpu.SemaphoreType.DMA] * (2 * DEPTH)
        ),
    )


def _sc_deg(dst):
    return _sc_deg_kernel()(dst)


def _sc_agg(g, src3, dst3, zacc):
    return _sc_agg_kernel()(g, src3, dst3, zacc)


def _elu(v):
    return jnp.where(v > 0, v, jnp.exp(v) - 1.0)


def _time_mlp(tt, tw1, tb1, tw2, tb2):
    half = 64
    k = -(math.log(10000.0) / (half - 1))
    io = lax.broadcasted_iota(jnp.int32, (1, half), 1).astype(_f32)
    freq = jnp.exp(io * k)
    emb = tt * freq
    temb = jnp.concatenate([jnp.sin(emb), jnp.cos(emb)], axis=-1)
    hvec = _elu(jnp.dot(temb, tw1, preferred_element_type=_f32) + tb1)
    return jnp.dot(hvec, tw2, preferred_element_type=_f32) + tb2


def _dinv_of(degt):
    return lax.rsqrt(jnp.sum(degt, axis=1, keepdims=True) + 2.0)


def _tc0_body(x_ref, qy_ref, wa_ref, wb_ref, dg_ref, g_ref):
    dinv = _dinv_of(dg_ref[...])
    hw = (jnp.dot(x_ref[...], wa_ref[...], preferred_element_type=_f32)
          + jnp.dot(qy_ref[...], wb_ref[...], preferred_element_type=_f32))
    g_ref[...] = hw * dinv


def _tc_layer_body(tt_ref, q0_ref, q1_ref, g_ref, qy_ref, dg_ref,
                   wa_ref, wb_ref, b_ref, tw1_ref, tb1_ref, tw2_ref, tb2_ref,
                   out_ref):
    dinv = _dinv_of(dg_ref[...])
    tv = _time_mlp(tt_ref[...], tw1_ref[...], tb1_ref[...], tw2_ref[...], tb2_ref[...])
    x1 = dinv * (q0_ref[...] + q1_ref[...] + 2.0 * g_ref[...]) + b_ref[...] + tv
    x1 = jnp.maximum(x1, 0.0)
    hw = (jnp.dot(x1, wa_ref[...], preferred_element_type=_f32)
          + jnp.dot(qy_ref[...], wb_ref[...], preferred_element_type=_f32))
    out_ref[...] = hw * dinv


def _tc_head_body(tt_ref, q0_ref, q1_ref, g_ref, qy_ref, dg_ref,
                  fa_ref, fb_ref, b_ref, fb1_ref, fw2_ref, fb2_ref,
                  tw1_ref, tb1_ref, tw2_ref, tb2_ref, out_ref):
    dinv = _dinv_of(dg_ref[...])
    tv = _time_mlp(tt_ref[...], tw1_ref[...], tb1_ref[...], tw2_ref[...], tb2_ref[...])
    x2 = dinv * (q0_ref[...] + q1_ref[...] + 2.0 * g_ref[...]) + b_ref[...] + tv
    x2 = jnp.maximum(x2, 0.0)
    z = _elu(jnp.dot(x2, fa_ref[...], preferred_element_type=_f32)
             + jnp.dot(qy_ref[...], fb_ref[...], preferred_element_type=_f32)
             + fb1_ref[...])
    out_ref[...] = jnp.dot(z, fw2_ref[...], preferred_element_type=_f32) + fb2_ref[...]


_R = 1000
_G = N // _R


def _row_spec(w):
    return pl.BlockSpec((_R, w), lambda i: (i, 0))


def _full_spec(r, c):
    return pl.BlockSpec((r, c), lambda i: (0, 0))


def _tc0(x, qyp, wa, wb, degt):
    return pl.pallas_call(
        _tc0_body,
        grid=(_G,),
        in_specs=[_row_spec(H), _row_spec(16), _full_spec(H, H), _full_spec(16, H),
                  _row_spec(NTILES)],
        out_specs=_row_spec(H),
        out_shape=jax.ShapeDtypeStruct((N, H), _f32),
    )(x, qyp, wa, wb, degt)


def _tc_layer(tt, q0, q1, g, qyp, degt, wa, wb, b, tw1, tb1, tw2, tb2):
    return pl.pallas_call(
        _tc_layer_body,
        grid=(_G,),
        in_specs=[_full_spec(1, 1), _row_spec(H), _row_spec(H), _row_spec(H),
                  _row_spec(16), _row_spec(NTILES),
                  _full_spec(H, H), _full_spec(16, H), _full_spec(1, H),
                  _full_spec(H, H), _full_spec(1, H), _full_spec(H, H), _full_spec(1, H)],
        out_specs=_row_spec(H),
        out_shape=jax.ShapeDtypeStruct((N, H), _f32),
    )(tt, q0, q1, g, qyp, degt, wa, wb, b, tw1, tb1, tw2, tb2)


def _tc_head(tt, q0, q1, g, qyp, degt, fa, fb, b, fb1, fw2, fb2, tw1, tb1, tw2, tb2):
    F2 = 2 * (H + 10)
    return pl.pallas_call(
        _tc_head_body,
        grid=(_G,),
        in_specs=[_full_spec(1, 1), _row_spec(H), _row_spec(H), _row_spec(H),
                  _row_spec(16), _row_spec(NTILES),
                  _full_spec(H, F2), _full_spec(16, F2), _full_spec(1, H),
                  _full_spec(1, F2), _full_spec(F2, 10), _full_spec(1, 10),
                  _full_spec(H, H), _full_spec(1, H), _full_spec(H, H), _full_spec(1, H)],
        out_specs=_row_spec(10),
        out_shape=jax.ShapeDtypeStruct((N, 10), _f32),
    )(tt, q0, q1, g, qyp, degt, fa, fb, b, fb1, fw2, fb2, tw1, tb1, tw2, tb2)


def kernel(x, q_Y_sample, adj, t, num_steps, W0, b0, W1, b1,
           tW1, tb1, tW2, tb2, fW1, fb1, fW2, fb2):
    src, dst = adj[0], adj[1]
    src3 = jnp.pad(src.reshape(NTILES, EPT), ((0, 0), (0, EPP - EPT)),
                   constant_values=0).reshape(-1)
    dst3 = jnp.pad(dst.reshape(NTILES, EPT), ((0, 0), (0, EPP - EPT)),
                   constant_values=N).reshape(-1)

    qyp = jnp.pad(q_Y_sample, ((0, 0), (0, 6)))
    W0a, W0b = W0[:H], jnp.pad(W0[H:], ((0, 6), (0, 0)))
    W1a, W1b = W1[:H], jnp.pad(W1[H:], ((0, 6), (0, 0)))
    fW1a, fW1b = fW1[:H], jnp.pad(fW1[H:], ((0, 6), (0, 0)))
    tt = ((t / num_steps) * num_steps * 4.0).astype(_f32).reshape(1, 1)
    zacc = jnp.zeros((NA, H), _f32)

    degt = _sc_deg(dst).reshape(NTILES, N).T

    g0 = _tc0(x, qyp, W0a, W0b, degt)
    q0, q1 = _sc_agg(g0, src3, dst3, zacc)
    g1 = _tc_layer(tt, q0, q1, g0, qyp, degt, W1a, W1b,
                   b0.reshape(1, -1), tW1, tb1.reshape(1, -1), tW2, tb2.reshape(1, -1))
    q0b, q1b = _sc_agg(g1, src3, dst3, zacc)
    pred = _tc_head(tt, q0b, q1b, g1, qyp, degt, fW1a, fW1b,
                    b1.reshape(1, -1), fb1.reshape(1, -1), fW2, fb2.reshape(1, -1),
                    tW1, tb1.reshape(1, -1), tW2, tb2.reshape(1, -1))
    return pred


# speedup vs baseline: 1.4366x; 1.4366x over previous
import functools
import math

import jax
import jax.numpy as jnp
from jax import lax
from jax.experimental import pallas as pl
from jax.experimental.pallas import tpu as pltpu
from jax.experimental.pallas import tpu_sc as plsc

N = 10000
E = 320000
NP = 10240
H = 128
CH = 128
NTILES = 32
EPT_REAL = E // NTILES
NCH = -(-EPT_REAL // CH)
EPT = NCH * CH
EPAD = EPT * NTILES
RT = NP // 16

_f32 = jnp.float32


def _sc_mesh():
    return plsc.VectorSubcoreMesh(core_axis_name="c", subcore_axis_name="s",
                                  num_cores=2, num_subcores=16)


def _sc_deg_body(dst_hbm, out_hbm, didx, cnt, sem):
    cid = lax.axis_index("c")
    sid = lax.axis_index("s")
    wid = cid * 16 + sid

    def zrow(i, c):
        cnt[pl.ds(i * 16, 16)] = jnp.zeros((16,), _f32)
        return c

    lax.fori_loop(0, NP // 16, zrow, 0)
    pltpu.sync_copy(dst_hbm.at[pl.ds(wid * EPT, EPT)], didx)
    ones = jnp.ones((16,), _f32)

    def grp(g, c):
        idx = didx[pl.ds(g * 16, 16)]
        plsc.addupdate_scatter(cnt, [idx], ones)
        return c

    lax.fori_loop(0, EPT // 16, grp, 0)
    pltpu.sync_copy(cnt, out_hbm.at[pl.ds(wid * NP, NP)])


def _sc_agg_body(g_hbm, src_hbm, dst_hbm, zero_hbm, out_hbm, sidx, didx, rows, acc, sem):
    cid = lax.axis_index("c")
    sid = lax.axis_index("s")
    wid = cid * 16 + sid
    r0 = sid * RT
    pltpu.sync_copy(zero_hbm.at[pl.ds(r0, RT)], acc.at[pl.ds(r0, RT)])
    plsc.subcore_barrier()

    def chunk(j, carry):
        off = pl.multiple_of(wid * EPT + j * CH, CH)
        pltpu.sync_copy(src_hbm.at[pl.ds(off, CH)], sidx)
        pltpu.sync_copy(dst_hbm.at[pl.ds(off, CH)], didx)
        pltpu.async_copy(g_hbm.at[sidx], rows, sem).wait()
        pltpu.sync_copy(rows, acc.at[didx], add=True)
        return carry

    lax.fori_loop(0, NCH, chunk, 0)
    plsc.subcore_barrier()
    pltpu.sync_copy(acc.at[pl.ds(r0, RT)], out_hbm.at[pl.ds(cid * NP + r0, RT)])


@functools.lru_cache(maxsize=None)
def _sc_deg_kernel():
    return pl.kernel(
        _sc_deg_body,
        out_type=jax.ShapeDtypeStruct((NTILES * NP,), _f32),
        mesh=_sc_mesh(),
        compiler_params=pltpu.CompilerParams(needs_layout_passes=False),
        scratch_types=[
            pltpu.VMEM((EPT,), jnp.int32),
            pltpu.VMEM((NP,), _f32),
            pltpu.SemaphoreType.DMA,
        ],
    )


@functools.lru_cache(maxsize=None)
def _sc_agg_kernel():
    return pl.kernel(
        _sc_agg_body,
        out_type=jax.ShapeDtypeStruct((2 * NP, H), _f32),
        mesh=_sc_mesh(),
        scratch_types=[
            pltpu.VMEM((CH,), jnp.int32),
            pltpu.VMEM((CH,), jnp.int32),
            pltpu.VMEM((CH, H), _f32),
            pltpu.VMEM_SHARED((NP, H), _f32),
            pltpu.SemaphoreType.DMA,
        ],
    )


def _sc_deg(dstp):
    return _sc_deg_kernel()(dstp)


def _sc_agg(g, srcp, dstp, z128):
    return _sc_agg_kernel()(g, srcp, dstp, z128)


def _elu(v):
    return jnp.where(v > 0, v, jnp.exp(v) - 1.0)


def _time_mlp(tt, tw1, tb1, tw2, tb2):
    half = 64
    k = -(math.log(10000.0) / (half - 1))
    io = lax.broadcasted_iota(jnp.int32, (1, half), 1).astype(_f32)
    freq = jnp.exp(io * k)
    emb = tt * freq
    temb = jnp.concatenate([jnp.sin(emb), jnp.cos(emb)], axis=-1)
    hvec = _elu(jnp.dot(temb, tw1, preferred_element_type=_f32) + tb1)
    return jnp.dot(hvec, tw2, preferred_element_type=_f32) + tb2


def _dinv_of(degt):
    return lax.rsqrt(jnp.sum(degt, axis=1, keepdims=True) + 2.0)


def _tc0_body(x_ref, qy_ref, wa_ref, wb_ref, dg_ref, g_ref):
    dinv = _dinv_of(dg_ref[...])
    hw = (jnp.dot(x_ref[...], wa_ref[...], preferred_element_type=_f32)
          + jnp.dot(qy_ref[...], wb_ref[...], preferred_element_type=_f32))
    g_ref[...] = hw * dinv


def _tc_layer_body(tt_ref, q0_ref, q1_ref, g_ref, qy_ref, dg_ref,
                   wa_ref, wb_ref, b_ref, tw1_ref, tb1_ref, tw2_ref, tb2_ref,
                   out_ref):
    dinv = _dinv_of(dg_ref[...])
    tv = _time_mlp(tt_ref[...], tw1_ref[...], tb1_ref[...], tw2_ref[...], tb2_ref[...])
    x1 = dinv * (q0_ref[...] + q1_ref[...] + 2.0 * g_ref[...]) + b_ref[...] + tv
    x1 = jnp.maximum(x1, 0.0)
    hw = (jnp.dot(x1, wa_ref[...], preferred_element_type=_f32)
          + jnp.dot(qy_ref[...], wb_ref[...], preferred_element_type=_f32))
    out_ref[...] = hw * dinv


def _tc_head_body(tt_ref, q0_ref, q1_ref, g_ref, qy_ref, dg_ref,
                  fa_ref, fb_ref, b_ref, fb1_ref, fw2_ref, fb2_ref,
                  tw1_ref, tb1_ref, tw2_ref, tb2_ref, out_ref):
    dinv = _dinv_of(dg_ref[...])
    tv = _time_mlp(tt_ref[...], tw1_ref[...], tb1_ref[...], tw2_ref[...], tb2_ref[...])
    x2 = dinv * (q0_ref[...] + q1_ref[...] + 2.0 * g_ref[...]) + b_ref[...] + tv
    x2 = jnp.maximum(x2, 0.0)
    z = _elu(jnp.dot(x2, fa_ref[...], preferred_element_type=_f32)
             + jnp.dot(qy_ref[...], fb_ref[...], preferred_element_type=_f32)
             + fb1_ref[...])
    out_ref[...] = jnp.dot(z, fw2_ref[...], preferred_element_type=_f32) + fb2_ref[...]


_R = 1024
_G = NP // _R


def _row_spec(w):
    return pl.BlockSpec((_R, w), lambda i: (i, 0))


def _full_spec(r, c):
    return pl.BlockSpec((r, c), lambda i: (0, 0))


def _tc0(xp, qyp, wa, wb, degt):
    return pl.pallas_call(
        _tc0_body,
        grid=(_G,),
        in_specs=[_row_spec(H), _row_spec(16), _full_spec(H, H), _full_spec(16, H),
                  _row_spec(NTILES)],
        out_specs=_row_spec(H),
        out_shape=jax.ShapeDtypeStruct((NP, H), _f32),
    )(xp, qyp, wa, wb, degt)


def _tc_layer(tt, q0, q1, g, qyp, degt, wa, wb, b, tw1, tb1, tw2, tb2):
    return pl.pallas_call(
        _tc_layer_body,
        grid=(_G,),
        in_specs=[_full_spec(1, 1), _row_spec(H), _row_spec(H), _row_spec(H),
                  _row_spec(16), _row_spec(NTILES),
                  _full_spec(H, H), _full_spec(16, H), _full_spec(1, H),
                  _full_spec(H, H), _full_spec(1, H), _full_spec(H, H), _full_spec(1, H)],
        out_specs=_row_spec(H),
        out_shape=jax.ShapeDtypeStruct((NP, H), _f32),
    )(tt, q0, q1, g, qyp, degt, wa, wb, b, tw1, tb1, tw2, tb2)


def _tc_head(tt, q0, q1, g, qyp, degt, fa, fb, b, fb1, fw2, fb2, tw1, tb1, tw2, tb2):
    F2 = 2 * (H + 10)
    return pl.pallas_call(
        _tc_head_body,
        grid=(_G,),
        in_specs=[_full_spec(1, 1), _row_spec(H), _row_spec(H), _row_spec(H),
                  _row_spec(16), _row_spec(NTILES),
                  _full_spec(H, F2), _full_spec(16, F2), _full_spec(1, H),
                  _full_spec(1, F2), _full_spec(F2, 10), _full_spec(1, 10),
                  _full_spec(H, H), _full_spec(1, H), _full_spec(H, H), _full_spec(1, H)],
        out_specs=_row_spec(10),
        out_shape=jax.ShapeDtypeStruct((NP, 10), _f32),
    )(tt, q0, q1, g, qyp, degt, fa, fb, b, fb1, fw2, fb2, tw1, tb1, tw2, tb2)


def kernel(x, q_Y_sample, adj, t, num_steps, W0, b0, W1, b1,
           tW1, tb1, tW2, tb2, fW1, fb1, fW2, fb2):
    src, dst = adj[0], adj[1]
    srcp = jnp.pad(src.reshape(NTILES, EPT_REAL), ((0, 0), (0, EPT - EPT_REAL)),
                   constant_values=0).reshape(-1)
    dstp = jnp.pad(dst.reshape(NTILES, EPT_REAL), ((0, 0), (0, EPT - EPT_REAL)),
                   constant_values=N).reshape(-1)

    xp = jnp.pad(x, ((0, NP - N), (0, 0)))
    qyp = jnp.pad(q_Y_sample, ((0, NP - N), (0, 6)))
    W0a, W0b = W0[:H], jnp.pad(W0[H:], ((0, 6), (0, 0)))
    W1a, W1b = W1[:H], jnp.pad(W1[H:], ((0, 6), (0, 0)))
    fW1a, fW1b = fW1[:H], jnp.pad(fW1[H:], ((0, 6), (0, 0)))
    tt = ((t / num_steps) * num_steps * 4.0).astype(_f32).reshape(1, 1)
    z128 = jnp.zeros((NP, H), _f32)

    degt = _sc_deg(dstp).reshape(NTILES, NP).T

    g0 = _tc0(xp, qyp, W0a, W0b, degt)
    qq = _sc_agg(g0, srcp, dstp, z128)
    g1 = _tc_layer(tt, qq[:NP], qq[NP:], g0, qyp, degt, W1a, W1b,
                   b0.reshape(1, -1), tW1, tb1.reshape(1, -1), tW2, tb2.reshape(1, -1))
    qq2 = _sc_agg(g1, srcp, dstp, z128)
    pred = _tc_head(tt, qq2[:NP], qq2[NP:], g1, qyp, degt, fW1a, fW1b,
                    b1.reshape(1, -1), fb1.reshape(1, -1), fW2, fb2.reshape(1, -1),
                    tW1, tb1.reshape(1, -1), tW2, tb2.reshape(1, -1))
    return pred[:N]
